# baseline (speedup 1.0000x reference)
import jax
import jax.numpy as jnp
from jax.experimental import pallas as pl


def _select_body(mem_ref, pos_ref, mv_ref, out_ref):
    m = pos_ref[...][:, :, None] == 1
    out_ref[...] = jnp.where(m, mv_ref[...], mem_ref[...])


def kernel(memory, positions, memory_vectors):
    B, N, D = memory.shape
    BLK = 256
    grid = (N // BLK,)
    return pl.pallas_call(
        _select_body,
        grid=grid,
        in_specs=[
            pl.BlockSpec((B, BLK, D), lambda i: (0, i, 0)),
            pl.BlockSpec((B, BLK), lambda i: (0, i)),
            pl.BlockSpec((B, BLK, D), lambda i: (0, i, 0)),
        ],
        out_specs=pl.BlockSpec((B, BLK, D), lambda i: (0, i, 0)),
        out_shape=jax.ShapeDtypeStruct((B, N, D), jnp.float32),
    )(memory, positions, memory_vectors)


# speedup vs baseline: 1.0390x; 1.0390x over previous
import jax
import jax.numpy as jnp
from jax.experimental import pallas as pl


def _select_body(mem_ref, pos_ref, mv_ref, out_ref):
    m = pos_ref[...] == 1
    out_ref[...] = jnp.where(m, mv_ref[...], mem_ref[...])


def kernel(memory, positions, memory_vectors):
    B, N, D = memory.shape
    R = B * N
    mem2 = memory.reshape(R, D)
    mv2 = memory_vectors.reshape(R, D)
    pos2 = positions.reshape(R, 1)
    BLKR = 4096
    grid = (R // BLKR,)
    out = pl.pallas_call(
        _select_body,
        grid=grid,
        in_specs=[
            pl.BlockSpec((BLKR, D), lambda i: (i, 0)),
            pl.BlockSpec((BLKR, 1), lambda i: (i, 0)),
            pl.BlockSpec((BLKR, D), lambda i: (i, 0)),
        ],
        out_specs=pl.BlockSpec((BLKR, D), lambda i: (i, 0)),
        out_shape=jax.ShapeDtypeStruct((R, D), jnp.float32),
    )(mem2, pos2, mv2)
    return out.reshape(B, N, D)
